# Initial kernel scaffold; baseline (speedup 1.0000x reference)
import jax
import jax.numpy as jnp
from jax import lax
from jax.experimental import pallas as pl
from jax.experimental.pallas import tpu as pltpu
from jax.experimental.pallas import tpu_sc as plsc

_T = 26
_B = 1024
_ROWS = 100000
_D = 64
_L = 20

_NC = 2
_NS = 16
_NW = _NC * _NS

_CHUNK_BAGS = 32
_IDX_PER_CHUNK = _CHUNK_BAGS * _L
_N_CHUNKS = _T * _B // _CHUNK_BAGS
_CHUNKS_PER_W = _N_CHUNKS // _NW
_GSPLIT = 128
_NG = _IDX_PER_CHUNK // _GSPLIT
_BPC = _B // _CHUNK_BAGS


def _sc_body(idx_hbm, w_hbm, out_hbm, idx_v, rows_v, acc_v, sem):
    wid = lax.axis_index("s") * _NC + lax.axis_index("c")
    c0 = wid * _CHUNKS_PER_W

    def chunk_body(g, carry):
        c = c0 + g
        t = c // _BPC
        b0 = (c % _BPC) * _CHUNK_BAGS

        pltpu.sync_copy(idx_hbm.at[pl.ds(c * _NG, _NG)], idx_v)

        off = jnp.full((16,), t * _ROWS, dtype=jnp.int32)
        for r in range(_NG):
            for cc in range(_GSPLIT // 16):
                sl = (r, pl.ds(cc * 16, 16))
                idx_v[sl] = idx_v[sl] + off

        copies = [
            pltpu.async_copy(
                w_hbm.at[idx_v.at[j]],
                rows_v.at[pl.ds(j * _GSPLIT, _GSPLIT)],
                sem,
            )
            for j in range(_NG)
        ]
        for cp in copies:
            cp.wait()

        def bag_body(i, bc):
            r0 = i * _L
            accs = [rows_v[r0, pl.ds(cg * 16, 16)] for cg in range(4)]
            for l in range(1, _L):
                for cg in range(4):
                    accs[cg] = accs[cg] + rows_v[r0 + l, pl.ds(cg * 16, 16)]
            for cg in range(4):
                acc_v[i, pl.ds(cg * 16, 16)] = accs[cg]
            return bc

        lax.fori_loop(0, _CHUNK_BAGS, bag_body, 0)

        pltpu.sync_copy(
            acc_v, out_hbm.at[pl.ds(b0, _CHUNK_BAGS), pl.ds(t * _D, _D)]
        )
        return carry

    lax.fori_loop(0, _CHUNKS_PER_W, chunk_body, 0)


def kernel(indices, offsets, weights):
    del offsets
    idx2d = indices.reshape(_T * _B * _L // _GSPLIT, _GSPLIT)
    mesh = plsc.VectorSubcoreMesh(
        core_axis_name="c", subcore_axis_name="s", num_cores=_NC,
        num_subcores=_NS,
    )
    run = pl.kernel(
        _sc_body,
        out_type=jax.ShapeDtypeStruct((_B, _T * _D), jnp.float32),
        mesh=mesh,
        scratch_types=[
            pltpu.VMEM((_NG, _GSPLIT), jnp.int32),
            pltpu.VMEM((_IDX_PER_CHUNK, _D), jnp.float32),
            pltpu.VMEM((_CHUNK_BAGS, _D), jnp.float32),
            pltpu.SemaphoreType.DMA,
        ],
    )
    return run(idx2d, weights)


# speedup vs baseline: 2.9059x; 2.9059x over previous
import jax
import jax.numpy as jnp
from jax import lax
from jax.experimental import pallas as pl
from jax.experimental.pallas import tpu as pltpu
from jax.experimental.pallas import tpu_sc as plsc

_T = 26
_B = 1024
_ROWS = 100000
_D = 64
_L = 20

_NC = 2
_NS = 16
_NW = _NC * _NS

_CHUNK_BAGS = 32
_IDX_PER_CHUNK = _CHUNK_BAGS * _L
_N_CHUNKS = _T * _B // _CHUNK_BAGS
_CHUNKS_PER_W = _N_CHUNKS // _NW
_GSPLIT = 128
_NG = _IDX_PER_CHUNK // _GSPLIT
_BPC = _B // _CHUNK_BAGS


def _sc_body(idx_hbm, w_hbm, out_hbm, idx_v, rows_v, acc_v, sem, sem_o):
    wid = lax.axis_index("s") * _NC + lax.axis_index("c")
    c0 = wid * _CHUNKS_PER_W

    def chunk_body(g, carry):
        c = c0 + g
        t = c // _BPC
        b0 = (c % _BPC) * _CHUNK_BAGS

        icopies = [
            pltpu.async_copy(
                idx_hbm.at[pl.ds(c * _IDX_PER_CHUNK + j * _GSPLIT, _GSPLIT)],
                idx_v.at[j],
                sem,
            )
            for j in range(_NG)
        ]
        for cp in icopies:
            cp.wait()

        off = jnp.full((16,), t * _ROWS, dtype=jnp.int32)
        for r in range(_NG):
            for cc in range(_GSPLIT // 16):
                sl = (r, pl.ds(cc * 16, 16))
                idx_v[sl] = idx_v[sl] + off

        copies = [
            pltpu.async_copy(
                w_hbm.at[idx_v.at[j]],
                rows_v.at[pl.ds(j * _GSPLIT, _GSPLIT)],
                sem,
            )
            for j in range(_NG)
        ]
        for cp in copies:
            cp.wait()

        def bag_body(i, bc):
            r0 = i * _L
            accs = [rows_v[r0, pl.ds(cg * 16, 16)] for cg in range(4)]
            for l in range(1, _L):
                for cg in range(4):
                    accs[cg] = accs[cg] + rows_v[r0 + l, pl.ds(cg * 16, 16)]
            for cg in range(4):
                acc_v[i, pl.ds(cg * 16, 16)] = accs[cg]
            pltpu.async_copy(
                acc_v.at[i],
                out_hbm.at[pl.ds((b0 + i) * (_T * _D) + t * _D, _D)],
                sem_o,
            )
            return bc

        lax.fori_loop(0, _CHUNK_BAGS, bag_body, 0)

        for _ in range(_CHUNK_BAGS):
            pltpu.make_async_copy(
                acc_v.at[0], out_hbm.at[pl.ds(0, _D)], sem_o
            ).wait()
        return carry

    lax.fori_loop(0, _CHUNKS_PER_W, chunk_body, 0)


def kernel(indices, offsets, weights):
    del offsets
    mesh = plsc.VectorSubcoreMesh(
        core_axis_name="c", subcore_axis_name="s", num_cores=_NC,
        num_subcores=_NS,
    )
    run = pl.kernel(
        _sc_body,
        out_type=jax.ShapeDtypeStruct((_B * _T * _D,), jnp.float32),
        mesh=mesh,
        scratch_types=[
            pltpu.VMEM((_NG, _GSPLIT), jnp.int32),
            pltpu.VMEM((_IDX_PER_CHUNK, _D), jnp.float32),
            pltpu.VMEM((_CHUNK_BAGS, _D), jnp.float32),
            pltpu.SemaphoreType.DMA,
            pltpu.SemaphoreType.DMA,
        ],
        compiler_params=pltpu.CompilerParams(use_tc_tiling_on_sc=False),
    )
    return run(indices, weights).reshape(_B, _T * _D)


# speedup vs baseline: 3.0110x; 1.0362x over previous
import jax
import jax.numpy as jnp
from jax import lax
from jax.experimental import pallas as pl
from jax.experimental.pallas import tpu as pltpu
from jax.experimental.pallas import tpu_sc as plsc

_T = 26
_B = 1024
_ROWS = 100000
_D = 64
_L = 20

_NC = 2
_NS = 16
_NW = _NC * _NS

_CHUNK_BAGS = 32
_IDX_PER_CHUNK = _CHUNK_BAGS * _L
_N_CHUNKS = _T * _B // _CHUNK_BAGS
_CPW = _N_CHUNKS // _NW
_GSPLIT = 128
_NG = _IDX_PER_CHUNK // _GSPLIT
_BPC = _B // _CHUNK_BAGS
_ROWS_BYTES = _IDX_PER_CHUNK * _D * 4


def _sc_body(idx_hbm, w_hbm, out_hbm, idx_all, rows_a, rows_b,
             acc_a, acc_b, sem_a, sem_b, sem_o):
    wid = lax.axis_index("s") * _NC + lax.axis_index("c")
    c0 = wid * _CPW
    i0 = c0 * _IDX_PER_CHUNK

    rows_total = _CPW * _NG
    for r in range(rows_total):
        pltpu.async_copy(
            idx_hbm.at[pl.ds(i0 + r * _GSPLIT, _GSPLIT)], idx_all.at[r], sem_a
        )
    for r in range(rows_total):
        pltpu.make_async_copy(
            idx_hbm.at[pl.ds(0, _GSPLIT)], idx_all.at[0], sem_a
        ).wait()

    def adjust_body(k, carry):
        t = (c0 + k) // _BPC
        off = jnp.full((16,), t * _ROWS, dtype=jnp.int32)
        for j in range(_NG):
            for cc in range(_GSPLIT // 16):
                sl = (k * _NG + j, pl.ds(cc * 16, 16))
                idx_all[sl] = idx_all[sl] + off
        return carry

    lax.fori_loop(0, _CPW, adjust_body, 0)

    def fire5(k, buf, sem):
        for j in range(_NG):
            pltpu.async_copy(
                w_hbm.at[idx_all.at[k * _NG + j]],
                buf.at[pl.ds(j * _GSPLIT, _GSPLIT)],
                sem,
            )

    def drain_gather(buf, sem):
        pltpu.make_async_copy(
            w_hbm.at[pl.ds(0, _IDX_PER_CHUNK)], buf, sem
        ).wait()

    def drain_outs(acc):
        pltpu.make_async_copy(
            out_hbm.at[pl.ds(0, _CHUNK_BAGS * _D)], acc, sem_o
        ).wait()

    def pool(k, buf, acc):
        c = c0 + k
        t = c // _BPC
        b0 = (c % _BPC) * _CHUNK_BAGS

        def bag_body(i, bc):
            r0 = i * _L
            accs = [buf[r0, pl.ds(cg * 16, 16)] for cg in range(4)]
            for l in range(1, _L):
                for cg in range(4):
                    accs[cg] = accs[cg] + buf[r0 + l, pl.ds(cg * 16, 16)]
            for cg in range(4):
                acc[pl.ds(i * _D + cg * 16, 16)] = accs[cg]
            pltpu.async_copy(
                acc.at[pl.ds(i * _D, _D)],
                out_hbm.at[pl.ds((b0 + i) * (_T * _D) + t * _D, _D)],
                sem_o,
            )
            return bc

        lax.fori_loop(0, _CHUNK_BAGS, bag_body, 0)

    fire5(0, rows_a, sem_a)

    def iter_body(g, carry):
        fire5(2 * g + 1, rows_b, sem_b)

        @pl.when(g > 0)
        def _():
            drain_outs(acc_a)

        drain_gather(rows_a, sem_a)
        pool(2 * g, rows_a, acc_a)

        @pl.when(g < _CPW // 2 - 1)
        def _():
            fire5(2 * g + 2, rows_a, sem_a)

        @pl.when(g > 0)
        def _():
            drain_outs(acc_b)

        drain_gather(rows_b, sem_b)
        pool(2 * g + 1, rows_b, acc_b)
        return carry

    lax.fori_loop(0, _CPW // 2, iter_body, 0)
    drain_outs(acc_a)
    drain_outs(acc_b)


def kernel(indices, offsets, weights):
    del offsets
    mesh = plsc.VectorSubcoreMesh(
        core_axis_name="c", subcore_axis_name="s", num_cores=_NC,
        num_subcores=_NS,
    )
    run = pl.kernel(
        _sc_body,
        out_type=jax.ShapeDtypeStruct((_B * _T * _D,), jnp.float32),
        mesh=mesh,
        scratch_types=[
            pltpu.VMEM((_CPW * _NG, _GSPLIT), jnp.int32),
            pltpu.VMEM((_IDX_PER_CHUNK, _D), jnp.float32),
            pltpu.VMEM((_IDX_PER_CHUNK, _D), jnp.float32),
            pltpu.VMEM((_CHUNK_BAGS * _D,), jnp.float32),
            pltpu.VMEM((_CHUNK_BAGS * _D,), jnp.float32),
            pltpu.SemaphoreType.DMA,
            pltpu.SemaphoreType.DMA,
            pltpu.SemaphoreType.DMA,
        ],
        compiler_params=pltpu.CompilerParams(use_tc_tiling_on_sc=False),
    )
    return run(indices, weights).reshape(_B, _T * _D)


# speedup vs baseline: 3.7571x; 1.2478x over previous
import jax
import jax.numpy as jnp
from jax import lax
from jax.experimental import pallas as pl
from jax.experimental.pallas import tpu as pltpu
from jax.experimental.pallas import tpu_sc as plsc

_T = 26
_B = 1024
_ROWS = 100000
_D = 64
_L = 20

_NC = 2
_NS = 16
_NW = _NC * _NS

_CHUNK_BAGS = 16
_IDX_PER_CHUNK = _CHUNK_BAGS * _L
_N_CHUNKS = _T * _B // _CHUNK_BAGS
_CPW = _N_CHUNKS // _NW
_GL = 64
_NG = _IDX_PER_CHUNK // _GL
_BPC = _B // _CHUNK_BAGS

_NBR = 1024
_HALF = _T * _ROWS // 2
_SHIFT = (_HALF // _NBR) * _NBR
_TC_GRID = 1271
_PAIR_ROWS = _TC_GRID * _NBR


def _tc_relayout_body(lo_ref, hi_ref, out_ref):
    out_ref[:, 0:_D] = lo_ref[...].T
    out_ref[:, _D:2 * _D] = hi_ref[...].T


def _sc_body(idx_hbm, w_hbm, out_hbm, idx_all, rows_a, rows_b,
             acc_a, acc_b, sem_a, sem_b, sem_o):
    wid = lax.axis_index("s") * _NC + lax.axis_index("c")
    c0 = wid * _CPW
    i0 = c0 * _IDX_PER_CHUNK

    rows_total = _CPW * _NG
    for r in range(rows_total):
        pltpu.async_copy(
            idx_hbm.at[pl.ds(i0 + r * _GL, _GL)], idx_all.at[r], sem_a
        )
    for r in range(rows_total):
        pltpu.make_async_copy(
            idx_hbm.at[pl.ds(0, _GL)], idx_all.at[0], sem_a
        ).wait()

    def adjust_body(k, carry):
        t = (c0 + k) // _BPC
        base = t * _ROWS - jnp.where(t >= _T // 2, _SHIFT, 0)
        off = jnp.full((16,), base, dtype=jnp.int32)
        for j in range(_NG):
            for cc in range(_GL // 16):
                sl = (k * _NG + j, pl.ds(cc * 16, 16))
                idx_all[sl] = idx_all[sl] + off
        return carry

    lax.fori_loop(0, _CPW, adjust_body, 0)

    def fire5(k, buf, sem):
        for j in range(_NG):
            pltpu.async_copy(
                w_hbm.at[idx_all.at[k * _NG + j]],
                buf.at[pl.ds(j * _GL, _GL)],
                sem,
            )

    def drain_gather(buf, sem):
        pltpu.make_async_copy(
            w_hbm.at[pl.ds(0, _IDX_PER_CHUNK)], buf, sem
        ).wait()

    def drain_outs(acc):
        pltpu.make_async_copy(
            out_hbm.at[pl.ds(0, _CHUNK_BAGS * _D)], acc, sem_o
        ).wait()

    def pool(k, buf, acc):
        c = c0 + k
        t = c // _BPC
        b0 = (c % _BPC) * _CHUNK_BAGS
        hb = jnp.where(t >= _T // 2, _D, 0)

        def bag_body(i, bc):
            r0 = i * _L
            accs = [buf[r0, pl.ds(hb + cg * 16, 16)] for cg in range(4)]
            for l in range(1, _L):
                for cg in range(4):
                    accs[cg] = accs[cg] + buf[r0 + l, pl.ds(hb + cg * 16, 16)]
            for cg in range(4):
                acc[pl.ds(i * _D + cg * 16, 16)] = accs[cg]
            pltpu.async_copy(
                acc.at[pl.ds(i * _D, _D)],
                out_hbm.at[pl.ds((b0 + i) * (_T * _D) + t * _D, _D)],
                sem_o,
            )
            return bc

        lax.fori_loop(0, _CHUNK_BAGS, bag_body, 0)

    fire5(0, rows_a, sem_a)

    def iter_body(g, carry):
        fire5(2 * g + 1, rows_b, sem_b)

        @pl.when(g > 0)
        def _():
            drain_outs(acc_a)

        drain_gather(rows_a, sem_a)
        pool(2 * g, rows_a, acc_a)

        @pl.when(g < _CPW // 2 - 1)
        def _():
            fire5(2 * g + 2, rows_a, sem_a)

        @pl.when(g > 0)
        def _():
            drain_outs(acc_b)

        drain_gather(rows_b, sem_b)
        pool(2 * g + 1, rows_b, acc_b)
        return carry

    lax.fori_loop(0, _CPW // 2, iter_body, 0)
    drain_outs(acc_a)
    drain_outs(acc_b)


def kernel(indices, offsets, weights):
    del offsets
    wt = weights.T

    w_pairs = pl.pallas_call(
        _tc_relayout_body,
        grid=(_TC_GRID,),
        in_specs=[
            pl.BlockSpec((_D, _NBR), lambda i: (0, i)),
            pl.BlockSpec((_D, _NBR), lambda i: (0, _SHIFT // _NBR + i)),
        ],
        out_specs=pl.BlockSpec((_NBR, 2 * _D), lambda i: (i, 0)),
        out_shape=jax.ShapeDtypeStruct((_PAIR_ROWS, 2 * _D), jnp.float32),
    )(wt, wt)

    mesh = plsc.VectorSubcoreMesh(
        core_axis_name="c", subcore_axis_name="s", num_cores=_NC,
        num_subcores=_NS,
    )
    run = pl.kernel(
        _sc_body,
        out_type=jax.ShapeDtypeStruct((_B * _T * _D,), jnp.float32),
        mesh=mesh,
        scratch_types=[
            pltpu.VMEM((_CPW * _NG, _GL), jnp.int32),
            pltpu.VMEM((_IDX_PER_CHUNK, 2 * _D), jnp.float32),
            pltpu.VMEM((_IDX_PER_CHUNK, 2 * _D), jnp.float32),
            pltpu.VMEM((_CHUNK_BAGS * _D,), jnp.float32),
            pltpu.VMEM((_CHUNK_BAGS * _D,), jnp.float32),
            pltpu.SemaphoreType.DMA,
            pltpu.SemaphoreType.DMA,
            pltpu.SemaphoreType.DMA,
        ],
        compiler_params=pltpu.CompilerParams(use_tc_tiling_on_sc=True),
    )
    return run(indices, w_pairs).reshape(_B, _T * _D)


# speedup vs baseline: 4.9620x; 1.3207x over previous
import jax
import jax.numpy as jnp
from jax import lax
from jax.experimental import pallas as pl
from jax.experimental.pallas import tpu as pltpu
from jax.experimental.pallas import tpu_sc as plsc

_T = 26
_B = 1024
_ROWS = 100000
_D = 64
_L = 20

_NC = 2
_NS = 16
_NW = _NC * _NS

_CHUNK_BAGS = 16
_IDX_PER_CHUNK = _CHUNK_BAGS * _L
_N_CHUNKS = _T * _B // _CHUNK_BAGS
_CPW = _N_CHUNKS // _NW
_GL = 64
_NG = _IDX_PER_CHUNK // _GL
_BPC = _B // _CHUNK_BAGS

_NBR = 2048
_HALF = _T * _ROWS // 2
_SHIFT = (_HALF // _NBR) * _NBR
_TC_GRID = (_T * _ROWS - 1 - _SHIFT) // _NBR + 1
_PAIR_ROWS = _TC_GRID * _NBR


def _tc_relayout_body(lo_ref, hi_ref, out_ref):
    row = lax.broadcasted_iota(jnp.int32, (_D, _D), 0)
    col = lax.broadcasted_iota(jnp.int32, (_D, _D), 1)
    eye = (row == col).astype(jnp.float32)
    dn = (((0,), (0,)), ((), ()))
    out_ref[:, 0:_D] = lax.dot_general(
        lo_ref[...], eye, dn, preferred_element_type=jnp.float32
    )
    out_ref[:, _D:2 * _D] = lax.dot_general(
        hi_ref[...], eye, dn, preferred_element_type=jnp.float32
    )


def _sc_body(idx_hbm, w_hbm, out_hbm, idx_all, rows_a, rows_b,
             acc_a, acc_b, sem_a, sem_b, sem_o):
    wid = lax.axis_index("s") * _NC + lax.axis_index("c")
    c0 = wid * _CPW
    i0 = c0 * _IDX_PER_CHUNK

    rows_total = _CPW * _NG
    for r in range(rows_total):
        pltpu.async_copy(
            idx_hbm.at[pl.ds(i0 + r * _GL, _GL)], idx_all.at[r], sem_a
        )
    for r in range(rows_total):
        pltpu.make_async_copy(
            idx_hbm.at[pl.ds(0, _GL)], idx_all.at[0], sem_a
        ).wait()

    def adjust_body(k, carry):
        t = (c0 + k) // _BPC
        base = t * _ROWS - jnp.where(t >= _T // 2, _SHIFT, 0)
        off = jnp.full((16,), base, dtype=jnp.int32)
        for j in range(_NG):
            for cc in range(_GL // 16):
                sl = (k * _NG + j, pl.ds(cc * 16, 16))
                idx_all[sl] = idx_all[sl] + off
        return carry

    lax.fori_loop(0, _CPW, adjust_body, 0)

    def fire5(k, buf, sem):
        for j in range(_NG):
            pltpu.async_copy(
                w_hbm.at[idx_all.at[k * _NG + j]],
                buf.at[pl.ds(j * _GL, _GL)],
                sem,
            )

    def drain_gather(buf, sem):
        pltpu.make_async_copy(
            w_hbm.at[pl.ds(0, _IDX_PER_CHUNK)], buf, sem
        ).wait()

    def drain_outs(acc):
        pltpu.make_async_copy(
            out_hbm.at[pl.ds(0, _CHUNK_BAGS * _D)], acc, sem_o
        ).wait()

    def pool(k, buf, acc):
        c = c0 + k
        t = c // _BPC
        b0 = (c % _BPC) * _CHUNK_BAGS
        hb = jnp.where(t >= _T // 2, _D, 0)

        def bag_body(i, bc):
            r0 = i * _L
            accs = [buf[r0, pl.ds(hb + cg * 16, 16)] for cg in range(4)]
            for l in range(1, _L):
                for cg in range(4):
                    accs[cg] = accs[cg] + buf[r0 + l, pl.ds(hb + cg * 16, 16)]
            for cg in range(4):
                acc[pl.ds(i * _D + cg * 16, 16)] = accs[cg]
            pltpu.async_copy(
                acc.at[pl.ds(i * _D, _D)],
                out_hbm.at[pl.ds((b0 + i) * (_T * _D) + t * _D, _D)],
                sem_o,
            )
            return bc

        lax.fori_loop(0, _CHUNK_BAGS, bag_body, 0)

    fire5(0, rows_a, sem_a)

    def iter_body(g, carry):
        fire5(2 * g + 1, rows_b, sem_b)

        @pl.when(g > 0)
        def _():
            drain_outs(acc_a)

        drain_gather(rows_a, sem_a)
        pool(2 * g, rows_a, acc_a)

        @pl.when(g < _CPW // 2 - 1)
        def _():
            fire5(2 * g + 2, rows_a, sem_a)

        @pl.when(g > 0)
        def _():
            drain_outs(acc_b)

        drain_gather(rows_b, sem_b)
        pool(2 * g + 1, rows_b, acc_b)
        return carry

    lax.fori_loop(0, _CPW // 2, iter_body, 0)
    drain_outs(acc_a)
    drain_outs(acc_b)


def kernel(indices, offsets, weights):
    del offsets
    wt = weights.T

    w_pairs = pl.pallas_call(
        _tc_relayout_body,
        grid=(_TC_GRID,),
        in_specs=[
            pl.BlockSpec((_D, _NBR), lambda i: (0, i)),
            pl.BlockSpec((_D, _NBR), lambda i: (0, _SHIFT // _NBR + i)),
        ],
        out_specs=pl.BlockSpec((_NBR, 2 * _D), lambda i: (i, 0)),
        out_shape=jax.ShapeDtypeStruct((_PAIR_ROWS, 2 * _D), jnp.float32),
    )(wt, wt)

    mesh = plsc.VectorSubcoreMesh(
        core_axis_name="c", subcore_axis_name="s", num_cores=_NC,
        num_subcores=_NS,
    )
    run = pl.kernel(
        _sc_body,
        out_type=jax.ShapeDtypeStruct((_B * _T * _D,), jnp.float32),
        mesh=mesh,
        scratch_types=[
            pltpu.VMEM((_CPW * _NG, _GL), jnp.int32),
            pltpu.VMEM((_IDX_PER_CHUNK, 2 * _D), jnp.float32),
            pltpu.VMEM((_IDX_PER_CHUNK, 2 * _D), jnp.float32),
            pltpu.VMEM((_CHUNK_BAGS * _D,), jnp.float32),
            pltpu.VMEM((_CHUNK_BAGS * _D,), jnp.float32),
            pltpu.SemaphoreType.DMA,
            pltpu.SemaphoreType.DMA,
            pltpu.SemaphoreType.DMA,
        ],
        compiler_params=pltpu.CompilerParams(use_tc_tiling_on_sc=True),
    )
    return run(indices, w_pairs).reshape(_B, _T * _D)


# speedup vs baseline: 4.9620x; 1.0000x over previous
import jax
import jax.numpy as jnp
from jax import lax
from jax.experimental import pallas as pl
from jax.experimental.pallas import tpu as pltpu
from jax.experimental.pallas import tpu_sc as plsc

_T = 26
_B = 1024
_ROWS = 100000
_D = 64
_L = 20

_NC = 2
_NS = 16
_NW = _NC * _NS

_CHUNK_BAGS = 16
_IDX_PER_CHUNK = _CHUNK_BAGS * _L
_N_CHUNKS = _T * _B // _CHUNK_BAGS
_CPW = _N_CHUNKS // _NW
_GL = 64
_NG = _IDX_PER_CHUNK // _GL
_BPC = _B // _CHUNK_BAGS

_NBR = 2048
_HALF = _T * _ROWS // 2
_SHIFT = (_HALF // _NBR) * _NBR
_TC_GRID = (_T * _ROWS - 1 - _SHIFT) // _NBR + 1
_PAIR_ROWS = _TC_GRID * _NBR


def _tc_relayout_body(lo_ref, hi_ref, out_ref):
    out_ref[:, 0:_D] = lo_ref[...].T
    out_ref[:, _D:2 * _D] = hi_ref[...].T


def _sc_body(idx_hbm, w_hbm, out_hbm, idx_all, rows_a, rows_b,
             acc_a, acc_b, sem_a, sem_b, sem_o):
    wid = lax.axis_index("s") * _NC + lax.axis_index("c")
    c0 = wid * _CPW
    i0 = c0 * _IDX_PER_CHUNK

    rows_total = _CPW * _NG
    for r in range(rows_total):
        pltpu.async_copy(
            idx_hbm.at[pl.ds(i0 + r * _GL, _GL)], idx_all.at[r], sem_a
        )
    for r in range(rows_total):
        pltpu.make_async_copy(
            idx_hbm.at[pl.ds(0, _GL)], idx_all.at[0], sem_a
        ).wait()

    def adjust_body(k, carry):
        t = (c0 + k) // _BPC
        base = t * _ROWS - jnp.where(t >= _T // 2, _SHIFT, 0)
        off = jnp.full((16,), base, dtype=jnp.int32)
        for j in range(_NG):
            for cc in range(_GL // 16):
                sl = (k * _NG + j, pl.ds(cc * 16, 16))
                idx_all[sl] = idx_all[sl] + off
        return carry

    lax.fori_loop(0, _CPW, adjust_body, 0)

    def fire5(k, buf, sem):
        for j in range(_NG):
            pltpu.async_copy(
                w_hbm.at[idx_all.at[k * _NG + j]],
                buf.at[pl.ds(j * _GL, _GL)],
                sem,
            )

    def drain_gather(buf, sem):
        pltpu.make_async_copy(
            w_hbm.at[pl.ds(0, _IDX_PER_CHUNK)], buf, sem
        ).wait()

    def drain_outs(acc):
        pltpu.make_async_copy(
            out_hbm.at[pl.ds(0, _CHUNK_BAGS * _D)], acc, sem_o
        ).wait()

    def pool(k, buf, acc):
        c = c0 + k
        t = c // _BPC
        b0 = (c % _BPC) * _CHUNK_BAGS
        hb = jnp.where(t >= _T // 2, _D, 0)

        def bag_body(i, bc):
            r0 = i * _L
            accs = [buf[r0, pl.ds(hb + cg * 16, 16)] for cg in range(4)]
            for l in range(1, _L):
                for cg in range(4):
                    accs[cg] = accs[cg] + buf[r0 + l, pl.ds(hb + cg * 16, 16)]
            for cg in range(4):
                acc[pl.ds(i * _D + cg * 16, 16)] = accs[cg]
            pltpu.async_copy(
                acc.at[pl.ds(i * _D, _D)],
                out_hbm.at[pl.ds((b0 + i) * (_T * _D) + t * _D, _D)],
                sem_o,
            )
            return bc

        lax.fori_loop(0, _CHUNK_BAGS, bag_body, 0)

    fire5(0, rows_a, sem_a)

    def iter_body(g, carry):
        fire5(2 * g + 1, rows_b, sem_b)

        @pl.when(g > 0)
        def _():
            drain_outs(acc_a)

        drain_gather(rows_a, sem_a)
        pool(2 * g, rows_a, acc_a)

        @pl.when(g < _CPW // 2 - 1)
        def _():
            fire5(2 * g + 2, rows_a, sem_a)

        @pl.when(g > 0)
        def _():
            drain_outs(acc_b)

        drain_gather(rows_b, sem_b)
        pool(2 * g + 1, rows_b, acc_b)
        return carry

    lax.fori_loop(0, _CPW // 2, iter_body, 0)
    drain_outs(acc_a)
    drain_outs(acc_b)


def kernel(indices, offsets, weights):
    del offsets
    wt = weights.T

    w_pairs = pl.pallas_call(
        _tc_relayout_body,
        grid=(_TC_GRID,),
        in_specs=[
            pl.BlockSpec((_D, _NBR), lambda i: (0, i)),
            pl.BlockSpec((_D, _NBR), lambda i: (0, _SHIFT // _NBR + i)),
        ],
        out_specs=pl.BlockSpec((_NBR, 2 * _D), lambda i: (i, 0)),
        out_shape=jax.ShapeDtypeStruct((_PAIR_ROWS, 2 * _D), jnp.float32),
    )(wt, wt)

    mesh = plsc.VectorSubcoreMesh(
        core_axis_name="c", subcore_axis_name="s", num_cores=_NC,
        num_subcores=_NS,
    )
    run = pl.kernel(
        _sc_body,
        out_type=jax.ShapeDtypeStruct((_B * _T * _D,), jnp.float32),
        mesh=mesh,
        scratch_types=[
            pltpu.VMEM((_CPW * _NG, _GL), jnp.int32),
            pltpu.VMEM((_IDX_PER_CHUNK, 2 * _D), jnp.float32),
            pltpu.VMEM((_IDX_PER_CHUNK, 2 * _D), jnp.float32),
            pltpu.VMEM((_CHUNK_BAGS * _D,), jnp.float32),
            pltpu.VMEM((_CHUNK_BAGS * _D,), jnp.float32),
            pltpu.SemaphoreType.DMA,
            pltpu.SemaphoreType.DMA,
            pltpu.SemaphoreType.DMA,
        ],
        compiler_params=pltpu.CompilerParams(use_tc_tiling_on_sc=True),
    )
    return run(indices, w_pairs).reshape(_B, _T * _D)


# speedup vs baseline: 6.0722x; 1.2237x over previous
import jax
import jax.numpy as jnp
from jax import lax
from jax.experimental import pallas as pl
from jax.experimental.pallas import tpu as pltpu
from jax.experimental.pallas import tpu_sc as plsc

_T = 26
_B = 1024
_ROWS = 100000
_D = 64
_L = 20

_NC = 2
_NS = 16
_NW = _NC * _NS

_CHUNK_BAGS = 16
_IDX_PER_CHUNK = _CHUNK_BAGS * _L
_N_CHUNKS = _T * _B // _CHUNK_BAGS
_CPW = _N_CHUNKS // _NW
_GL = 64
_NG = _IDX_PER_CHUNK // _GL
_BPC = _B // _CHUNK_BAGS

_NBR = 4096
_HALF = _T * _ROWS // 2
_SHIFT = (_HALF // _NBR) * _NBR
_TC_GRID = (_T * _ROWS - 1 - _SHIFT) // _NBR + 1
_PAIR_ROWS = _TC_GRID * _NBR


def _tc_relayout_body(lo_ref, hi_ref, out_ref):
    out_ref[...] = jnp.concatenate(
        [lo_ref[...].T, hi_ref[...].T], axis=-1
    )


def _sc_body(idx_hbm, w_hbm, out_hbm, idx_all, rows_a, rows_b,
             acc_a, acc_b, sem_a, sem_b, sem_o):
    wid = lax.axis_index("s") * _NC + lax.axis_index("c")
    c0 = wid * _CPW
    i0 = c0 * _IDX_PER_CHUNK

    rows_total = _CPW * _NG
    for r in range(rows_total):
        pltpu.async_copy(
            idx_hbm.at[pl.ds(i0 + r * _GL, _GL)], idx_all.at[r], sem_a
        )
    for r in range(rows_total):
        pltpu.make_async_copy(
            idx_hbm.at[pl.ds(0, _GL)], idx_all.at[0], sem_a
        ).wait()

    def adjust_body(k, carry):
        t = (c0 + k) // _BPC
        base = t * _ROWS - jnp.where(t >= _T // 2, _SHIFT, 0)
        off = jnp.full((16,), base, dtype=jnp.int32)
        for j in range(_NG):
            for cc in range(_GL // 16):
                sl = (k * _NG + j, pl.ds(cc * 16, 16))
                idx_all[sl] = idx_all[sl] + off
        return carry

    lax.fori_loop(0, _CPW, adjust_body, 0)

    def fire5(k, buf, sem):
        for j in range(_NG):
            pltpu.async_copy(
                w_hbm.at[idx_all.at[k * _NG + j]],
                buf.at[pl.ds(j * _GL, _GL)],
                sem,
            )

    def drain_gather(buf, sem):
        pltpu.make_async_copy(
            w_hbm.at[pl.ds(0, _IDX_PER_CHUNK)], buf, sem
        ).wait()

    def drain_outs(acc):
        pltpu.make_async_copy(
            out_hbm.at[pl.ds(0, _CHUNK_BAGS * _D)], acc, sem_o
        ).wait()

    def pool(k, buf, acc):
        c = c0 + k
        t = c // _BPC
        b0 = (c % _BPC) * _CHUNK_BAGS
        hb = jnp.where(t >= _T // 2, _D, 0)

        def bag_body(i, bc):
            r0 = i * _L
            accs = [buf[r0, pl.ds(hb + cg * 16, 16)] for cg in range(4)]
            for l in range(1, _L):
                for cg in range(4):
                    accs[cg] = accs[cg] + buf[r0 + l, pl.ds(hb + cg * 16, 16)]
            for cg in range(4):
                acc[pl.ds(i * _D + cg * 16, 16)] = accs[cg]
            pltpu.async_copy(
                acc.at[pl.ds(i * _D, _D)],
                out_hbm.at[pl.ds((b0 + i) * (_T * _D) + t * _D, _D)],
                sem_o,
            )
            return bc

        lax.fori_loop(0, _CHUNK_BAGS, bag_body, 0)

    fire5(0, rows_a, sem_a)

    def iter_body(g, carry):
        fire5(2 * g + 1, rows_b, sem_b)

        @pl.when(g > 0)
        def _():
            drain_outs(acc_a)

        drain_gather(rows_a, sem_a)
        pool(2 * g, rows_a, acc_a)

        @pl.when(g < _CPW // 2 - 1)
        def _():
            fire5(2 * g + 2, rows_a, sem_a)

        @pl.when(g > 0)
        def _():
            drain_outs(acc_b)

        drain_gather(rows_b, sem_b)
        pool(2 * g + 1, rows_b, acc_b)
        return carry

    lax.fori_loop(0, _CPW // 2, iter_body, 0)
    drain_outs(acc_a)
    drain_outs(acc_b)


def kernel(indices, offsets, weights):
    del offsets
    wt = weights.T

    w_pairs = pl.pallas_call(
        _tc_relayout_body,
        grid=(_TC_GRID,),
        in_specs=[
            pl.BlockSpec((_D, _NBR), lambda i: (0, i)),
            pl.BlockSpec((_D, _NBR), lambda i: (0, _SHIFT // _NBR + i)),
        ],
        out_specs=pl.BlockSpec((_NBR, 2 * _D), lambda i: (i, 0)),
        out_shape=jax.ShapeDtypeStruct((_PAIR_ROWS, 2 * _D), jnp.float32),
    )(wt, wt)

    mesh = plsc.VectorSubcoreMesh(
        core_axis_name="c", subcore_axis_name="s", num_cores=_NC,
        num_subcores=_NS,
    )
    run = pl.kernel(
        _sc_body,
        out_type=jax.ShapeDtypeStruct((_B * _T * _D,), jnp.float32),
        mesh=mesh,
        scratch_types=[
            pltpu.VMEM((_CPW * _NG, _GL), jnp.int32),
            pltpu.VMEM((_IDX_PER_CHUNK, 2 * _D), jnp.float32),
            pltpu.VMEM((_IDX_PER_CHUNK, 2 * _D), jnp.float32),
            pltpu.VMEM((_CHUNK_BAGS * _D,), jnp.float32),
            pltpu.VMEM((_CHUNK_BAGS * _D,), jnp.float32),
            pltpu.SemaphoreType.DMA,
            pltpu.SemaphoreType.DMA,
            pltpu.SemaphoreType.DMA,
        ],
        compiler_params=pltpu.CompilerParams(use_tc_tiling_on_sc=True),
    )
    return run(indices, w_pairs).reshape(_B, _T * _D)


# speedup vs baseline: 6.8063x; 1.1209x over previous
import jax
import jax.numpy as jnp
from jax import lax
from jax.experimental import pallas as pl
from jax.experimental.pallas import tpu as pltpu
from jax.experimental.pallas import tpu_sc as plsc

_T = 26
_B = 1024
_ROWS = 100000
_D = 64
_L = 20

_NC = 2
_NS = 16
_NW = _NC * _NS

_CHUNK_BAGS = 16
_IDX_PER_CHUNK = _CHUNK_BAGS * _L
_N_CHUNKS = _T * _B // _CHUNK_BAGS
_CPW = _N_CHUNKS // _NW
_GL = 64
_NG = _IDX_PER_CHUNK // _GL
_BPC = _B // _CHUNK_BAGS

_NBR = 8192
_HALF = _T * _ROWS // 2
_SHIFT = (_HALF // _NBR) * _NBR
_TC_GRID = (_T * _ROWS - 1 - _SHIFT) // _NBR + 1
_PAIR_ROWS = _TC_GRID * _NBR


def _tc_relayout_body(lo_ref, hi_ref, out_ref):
    out_ref[...] = jnp.concatenate(
        [lo_ref[...].T, hi_ref[...].T], axis=-1
    )


def _sc_body(idx_hbm, w_hbm, out_hbm, idx_all, rows_a, rows_b,
             acc_a, acc_b, sem_a, sem_b, sem_o):
    wid = lax.axis_index("s") * _NC + lax.axis_index("c")
    c0 = wid * _CPW
    i0 = c0 * _IDX_PER_CHUNK

    rows_total = _CPW * _NG
    for r in range(rows_total):
        pltpu.async_copy(
            idx_hbm.at[pl.ds(i0 + r * _GL, _GL)], idx_all.at[r], sem_a
        )
    for r in range(rows_total):
        pltpu.make_async_copy(
            idx_hbm.at[pl.ds(0, _GL)], idx_all.at[0], sem_a
        ).wait()

    def adjust_body(k, carry):
        t = (c0 + k) // _BPC
        base = t * _ROWS - jnp.where(t >= _T // 2, _SHIFT, 0)
        off = jnp.full((16,), base, dtype=jnp.int32)
        for j in range(_NG):
            for cc in range(_GL // 16):
                sl = (k * _NG + j, pl.ds(cc * 16, 16))
                idx_all[sl] = idx_all[sl] + off
        return carry

    lax.fori_loop(0, _CPW, adjust_body, 0)

    def fire5(k, buf, sem):
        for j in range(_NG):
            pltpu.async_copy(
                w_hbm.at[idx_all.at[k * _NG + j]],
                buf.at[pl.ds(j * _GL, _GL)],
                sem,
            )

    def drain_gather(buf, sem):
        pltpu.make_async_copy(
            w_hbm.at[pl.ds(0, _IDX_PER_CHUNK)], buf, sem
        ).wait()

    def drain_outs(acc):
        pltpu.make_async_copy(
            out_hbm.at[pl.ds(0, _CHUNK_BAGS * _D)], acc, sem_o
        ).wait()

    def pool(k, buf, acc):
        c = c0 + k
        t = c // _BPC
        b0 = (c % _BPC) * _CHUNK_BAGS
        hb = jnp.where(t >= _T // 2, _D, 0)

        def bag_body(i, bc):
            r0 = i * _L
            accs = [buf[r0, pl.ds(hb + cg * 16, 16)] for cg in range(4)]
            for l in range(1, _L):
                for cg in range(4):
                    accs[cg] = accs[cg] + buf[r0 + l, pl.ds(hb + cg * 16, 16)]
            for cg in range(4):
                acc[pl.ds(i * _D + cg * 16, 16)] = accs[cg]
            pltpu.async_copy(
                acc.at[pl.ds(i * _D, _D)],
                out_hbm.at[pl.ds((b0 + i) * (_T * _D) + t * _D, _D)],
                sem_o,
            )
            return bc

        lax.fori_loop(0, _CHUNK_BAGS, bag_body, 0)

    fire5(0, rows_a, sem_a)

    def iter_body(g, carry):
        fire5(2 * g + 1, rows_b, sem_b)

        @pl.when(g > 0)
        def _():
            drain_outs(acc_a)

        drain_gather(rows_a, sem_a)
        pool(2 * g, rows_a, acc_a)

        @pl.when(g < _CPW // 2 - 1)
        def _():
            fire5(2 * g + 2, rows_a, sem_a)

        @pl.when(g > 0)
        def _():
            drain_outs(acc_b)

        drain_gather(rows_b, sem_b)
        pool(2 * g + 1, rows_b, acc_b)
        return carry

    lax.fori_loop(0, _CPW // 2, iter_body, 0)
    drain_outs(acc_a)
    drain_outs(acc_b)


def kernel(indices, offsets, weights):
    del offsets
    wt = weights.T

    w_pairs = pl.pallas_call(
        _tc_relayout_body,
        grid=(_TC_GRID,),
        in_specs=[
            pl.BlockSpec((_D, _NBR), lambda i: (0, i)),
            pl.BlockSpec((_D, _NBR), lambda i: (0, _SHIFT // _NBR + i)),
        ],
        out_specs=pl.BlockSpec((_NBR, 2 * _D), lambda i: (i, 0)),
        out_shape=jax.ShapeDtypeStruct((_PAIR_ROWS, 2 * _D), jnp.float32),
    )(wt, wt)

    mesh = plsc.VectorSubcoreMesh(
        core_axis_name="c", subcore_axis_name="s", num_cores=_NC,
        num_subcores=_NS,
    )
    run = pl.kernel(
        _sc_body,
        out_type=jax.ShapeDtypeStruct((_B * _T * _D,), jnp.float32),
        mesh=mesh,
        scratch_types=[
            pltpu.VMEM((_CPW * _NG, _GL), jnp.int32),
            pltpu.VMEM((_IDX_PER_CHUNK, 2 * _D), jnp.float32),
            pltpu.VMEM((_IDX_PER_CHUNK, 2 * _D), jnp.float32),
            pltpu.VMEM((_CHUNK_BAGS * _D,), jnp.float32),
            pltpu.VMEM((_CHUNK_BAGS * _D,), jnp.float32),
            pltpu.SemaphoreType.DMA,
            pltpu.SemaphoreType.DMA,
            pltpu.SemaphoreType.DMA,
        ],
        compiler_params=pltpu.CompilerParams(use_tc_tiling_on_sc=True),
    )
    return run(indices, w_pairs).reshape(_B, _T * _D)


# speedup vs baseline: 7.2234x; 1.0613x over previous
import jax
import jax.numpy as jnp
from jax import lax
from jax.experimental import pallas as pl
from jax.experimental.pallas import tpu as pltpu
from jax.experimental.pallas import tpu_sc as plsc

_T = 26
_B = 1024
_ROWS = 100000
_D = 64
_L = 20

_NC = 2
_NS = 16
_NW = _NC * _NS

_CHUNK_BAGS = 16
_IDX_PER_CHUNK = _CHUNK_BAGS * _L
_N_CHUNKS = _T * _B // _CHUNK_BAGS
_CPW = _N_CHUNKS // _NW
_GL = 64
_NG = _IDX_PER_CHUNK // _GL
_BPC = _B // _CHUNK_BAGS

_NBR = 16384
_HALF = _T * _ROWS // 2
_SHIFT = (_HALF // _NBR) * _NBR
_TC_GRID = (_T * _ROWS - 1 - _SHIFT) // _NBR + 1
_PAIR_ROWS = _TC_GRID * _NBR


def _tc_relayout_body(lo_ref, hi_ref, out_ref):
    out_ref[...] = jnp.concatenate(
        [lo_ref[...].T, hi_ref[...].T], axis=-1
    )


def _sc_body(idx_hbm, w_hbm, out_hbm, idx_all, rows_a, rows_b,
             acc_a, acc_b, sem_a, sem_b, sem_o):
    wid = lax.axis_index("s") * _NC + lax.axis_index("c")
    c0 = wid * _CPW
    i0 = c0 * _IDX_PER_CHUNK

    rows_total = _CPW * _NG
    for r in range(rows_total):
        pltpu.async_copy(
            idx_hbm.at[pl.ds(i0 + r * _GL, _GL)], idx_all.at[r], sem_a
        )
    for r in range(rows_total):
        pltpu.make_async_copy(
            idx_hbm.at[pl.ds(0, _GL)], idx_all.at[0], sem_a
        ).wait()

    def adjust_body(k, carry):
        t = (c0 + k) // _BPC
        base = t * _ROWS - jnp.where(t >= _T // 2, _SHIFT, 0)
        off = jnp.full((16,), base, dtype=jnp.int32)
        for j in range(_NG):
            for cc in range(_GL // 16):
                sl = (k * _NG + j, pl.ds(cc * 16, 16))
                idx_all[sl] = idx_all[sl] + off
        return carry

    lax.fori_loop(0, _CPW, adjust_body, 0)

    def fire5(k, buf, sem):
        for j in range(_NG):
            pltpu.async_copy(
                w_hbm.at[idx_all.at[k * _NG + j]],
                buf.at[pl.ds(j * _GL, _GL)],
                sem,
            )

    def drain_gather(buf, sem):
        pltpu.make_async_copy(
            w_hbm.at[pl.ds(0, _IDX_PER_CHUNK)], buf, sem
        ).wait()

    def drain_outs(acc):
        pltpu.make_async_copy(
            out_hbm.at[pl.ds(0, _CHUNK_BAGS * _D)], acc, sem_o
        ).wait()

    def pool(k, buf, acc):
        c = c0 + k
        t = c // _BPC
        b0 = (c % _BPC) * _CHUNK_BAGS
        hb = jnp.where(t >= _T // 2, _D, 0)

        def bag_body(i, bc):
            r0 = i * _L
            accs = [buf[r0, pl.ds(hb + cg * 16, 16)] for cg in range(4)]
            for l in range(1, _L):
                for cg in range(4):
                    accs[cg] = accs[cg] + buf[r0 + l, pl.ds(hb + cg * 16, 16)]
            for cg in range(4):
                acc[pl.ds(i * _D + cg * 16, 16)] = accs[cg]
            pltpu.async_copy(
                acc.at[pl.ds(i * _D, _D)],
                out_hbm.at[pl.ds((b0 + i) * (_T * _D) + t * _D, _D)],
                sem_o,
            )
            return bc

        lax.fori_loop(0, _CHUNK_BAGS, bag_body, 0)

    fire5(0, rows_a, sem_a)

    def iter_body(g, carry):
        fire5(2 * g + 1, rows_b, sem_b)

        @pl.when(g > 0)
        def _():
            drain_outs(acc_a)

        drain_gather(rows_a, sem_a)
        pool(2 * g, rows_a, acc_a)

        @pl.when(g < _CPW // 2 - 1)
        def _():
            fire5(2 * g + 2, rows_a, sem_a)

        @pl.when(g > 0)
        def _():
            drain_outs(acc_b)

        drain_gather(rows_b, sem_b)
        pool(2 * g + 1, rows_b, acc_b)
        return carry

    lax.fori_loop(0, _CPW // 2, iter_body, 0)
    drain_outs(acc_a)
    drain_outs(acc_b)


def kernel(indices, offsets, weights):
    del offsets
    wt = weights.T

    w_pairs = pl.pallas_call(
        _tc_relayout_body,
        grid=(_TC_GRID,),
        in_specs=[
            pl.BlockSpec((_D, _NBR), lambda i: (0, i)),
            pl.BlockSpec((_D, _NBR), lambda i: (0, _SHIFT // _NBR + i)),
        ],
        out_specs=pl.BlockSpec((_NBR, 2 * _D), lambda i: (i, 0)),
        out_shape=jax.ShapeDtypeStruct((_PAIR_ROWS, 2 * _D), jnp.float32),
    )(wt, wt)

    mesh = plsc.VectorSubcoreMesh(
        core_axis_name="c", subcore_axis_name="s", num_cores=_NC,
        num_subcores=_NS,
    )
    run = pl.kernel(
        _sc_body,
        out_type=jax.ShapeDtypeStruct((_B * _T * _D,), jnp.float32),
        mesh=mesh,
        scratch_types=[
            pltpu.VMEM((_CPW * _NG, _GL), jnp.int32),
            pltpu.VMEM((_IDX_PER_CHUNK, 2 * _D), jnp.float32),
            pltpu.VMEM((_IDX_PER_CHUNK, 2 * _D), jnp.float32),
            pltpu.VMEM((_CHUNK_BAGS * _D,), jnp.float32),
            pltpu.VMEM((_CHUNK_BAGS * _D,), jnp.float32),
            pltpu.SemaphoreType.DMA,
            pltpu.SemaphoreType.DMA,
            pltpu.SemaphoreType.DMA,
        ],
        compiler_params=pltpu.CompilerParams(use_tc_tiling_on_sc=True),
    )
    return run(indices, w_pairs).reshape(_B, _T * _D)
